# baseline (speedup 1.0000x reference)
import functools

import jax
import jax.numpy as jnp
from jax import lax
from jax.experimental import pallas as pl
from jax.experimental.pallas import tpu as pltpu

_P = 2048
_BR = 16
_CC = 32
_OFFS = [(dr, dc) for dr in range(5) for dc in range(5)]


def _sample_points(h, w, n, points):
    key = jax.random.key(42)
    kx, ky = jax.random.split(key)
    xs = jnp.clip(w / 2.0 + (w / 4.0) * jax.random.normal(kx, (points,)), 0, w - 1)
    ys = jnp.clip(h / 2.0 + (h / 4.0) * jax.random.normal(ky, (points,)), 0, h - 1)
    SF = jnp.stack([xs, ys], axis=1)
    SFL = jnp.broadcast_to(SF[None], (n, points, 2)).astype(jnp.int32)
    return SFL


def _attn_kernel(alpha_ref, mask_ref, x1_ref, x2_ref, out_ref, px2_ref):
    s = pl.program_id(1)
    c_steps = x1_ref.shape[1] // _CC
    h = x2_ref.shape[2]
    w = x2_ref.shape[3]

    def _phase_cols(blk, dc):
        if dc == 0:
            return jnp.concatenate(
                [blk[:, :, 2:3], blk[:, :, 1:2], blk[:, :, 0:w - 2]], axis=2)
        if dc == 1:
            return jnp.concatenate([blk[:, :, 1:2], blk[:, :, 0:w - 1]], axis=2)
        if dc == 2:
            return blk
        if dc == 3:
            return jnp.concatenate([blk[:, :, 1:w], blk[:, :, w - 2:w - 1]], axis=2)
        return jnp.concatenate(
            [blk[:, :, 2:w], blk[:, :, w - 2:w - 1], blk[:, :, w - 3:w - 2]], axis=2)

    @pl.when(s == 0)
    def _build_pad():
        def pad_body(ci, _):
            cs = pl.ds(ci * _CC, _CC)
            blk = x2_ref[0, cs]
            for dc in range(5):
                px2_ref[dc, cs, 2:h + 2, :] = _phase_cols(blk, dc)
            return 0

        lax.fori_loop(0, c_steps, pad_body, 0)

        def row_body(ci, _):
            cs = pl.ds(ci * _CC, _CC)
            for dc in range(5):
                px2_ref[dc, cs, 0, :] = px2_ref[dc, cs, 4, :]
                px2_ref[dc, cs, 1, :] = px2_ref[dc, cs, 3, :]
                px2_ref[dc, cs, h + 2, :] = px2_ref[dc, cs, h, :]
                px2_ref[dc, cs, h + 3, :] = px2_ref[dc, cs, h - 1, :]
            return 0

        lax.fori_loop(0, c_steps, row_body, 0)

    r0 = s * _BR
    zero = jnp.zeros((_BR, w), dtype=jnp.float32)

    def e_body(ci, carry):
        cs = pl.ds(ci * _CC, _CC)
        x1c = x1_ref[0, cs, :, :].astype(jnp.bfloat16)
        new = list(carry)
        for dc in range(5):
            win = px2_ref[dc, cs, pl.ds(r0, _BR + 8), :]
            for dr in range(5):
                o = dr * 5 + dc
                sh = win[:, dr:dr + _BR, :]
                part = jnp.sum(x1c * sh, axis=0)
                new[o] = new[o] + part.astype(jnp.float32)
        return tuple(new)

    es = lax.fori_loop(0, c_steps, e_body, tuple(zero for _ in _OFFS))

    m = es[0]
    for o in range(1, 25):
        m = jnp.maximum(m, es[o])
    exps = [jnp.exp(e - m) for e in es]
    tot = exps[0]
    for o in range(1, 25):
        tot = tot + exps[o]
    inv = 1.0 / tot
    wts = [(e * inv).astype(jnp.bfloat16) for e in exps]

    alpha = alpha_ref[0]
    msk = mask_ref[...]

    def f_body(ci, _):
        cs = pl.ds(ci * _CC, _CC)
        x1c = x1_ref[0, cs, :, :]
        acc = jnp.zeros((x1c.shape[0], _BR, w), dtype=jnp.bfloat16)
        for dc in range(5):
            win = px2_ref[dc, cs, pl.ds(r0, _BR + 8), :]
            for dr in range(5):
                o = dr * 5 + dc
                sh = win[:, dr:dr + _BR, :]
                acc = acc + wts[o][None] * sh
        xf = x1c + msk[None] * (acc.astype(jnp.float32) - x1c)
        out_ref[0, cs, :, :] = x1c + alpha * xf
        return 0

    lax.fori_loop(0, c_steps, f_body, 0)


@jax.jit
def kernel(x1, x2, alpha):
    n, c, h, w = x1.shape
    SFL = _sample_points(h, w, n, _P)
    ph = SFL[0, :, 0]
    pw = SFL[0, :, 1]
    mask = jnp.zeros((h, w), jnp.float32).at[ph, pw].set(1.0)

    grid = (n, h // _BR)
    out = pl.pallas_call(
        _attn_kernel,
        grid=grid,
        in_specs=[
            pl.BlockSpec(memory_space=pltpu.SMEM),
            pl.BlockSpec((_BR, w), lambda b, s: (s, 0)),
            pl.BlockSpec((1, c, _BR, w), lambda b, s: (b, 0, s, 0)),
            pl.BlockSpec((1, c, h, w), lambda b, s: (b, 0, 0, 0)),
        ],
        out_specs=pl.BlockSpec((1, c, _BR, w), lambda b, s: (b, 0, s, 0)),
        out_shape=jax.ShapeDtypeStruct((n, c, h, w), jnp.float32),
        scratch_shapes=[pltpu.VMEM((5, c, h + 8, w), jnp.bfloat16)],
        compiler_params=pltpu.CompilerParams(
            dimension_semantics=("arbitrary", "arbitrary"),
        ),
    )(alpha, mask, x1, x2.astype(jnp.bfloat16))
    return (out, SFL)


# speedup vs baseline: 1.1424x; 1.1424x over previous
import functools

import jax
import jax.numpy as jnp
from jax import lax
from jax.experimental import pallas as pl
from jax.experimental.pallas import tpu as pltpu

_P = 2048
_BR = 32
_CC = 32
_OFFS = [(dr, dc) for dr in range(5) for dc in range(5)]


def _sample_points(h, w, n, points):
    key = jax.random.key(42)
    kx, ky = jax.random.split(key)
    xs = jnp.clip(w / 2.0 + (w / 4.0) * jax.random.normal(kx, (points,)), 0, w - 1)
    ys = jnp.clip(h / 2.0 + (h / 4.0) * jax.random.normal(ky, (points,)), 0, h - 1)
    SF = jnp.stack([xs, ys], axis=1)
    SFL = jnp.broadcast_to(SF[None], (n, points, 2)).astype(jnp.int32)
    return SFL


def _attn_kernel(alpha_ref, mask_ref, x1_ref, x2_ref, out_ref, px2_ref):
    s = pl.program_id(1)
    c_steps = x1_ref.shape[1] // _CC
    h = x2_ref.shape[2]
    w = x2_ref.shape[3]

    @pl.when(s == 0)
    def _build_pad():
        def pad_body(ci, _):
            cs = pl.ds(ci * _CC, _CC)
            blk = x2_ref[0, cs]
            px2_ref[cs, 2:h + 2, 2:w + 2] = blk
            left = jnp.concatenate([blk[:, :, 2:3], blk[:, :, 1:2]], axis=2)
            right = jnp.concatenate([blk[:, :, w - 2:w - 1], blk[:, :, w - 3:w - 2]], axis=2)
            px2_ref[cs, 2:h + 2, 0:2] = left
            px2_ref[cs, 2:h + 2, w + 2:w + 4] = right
            return 0

        lax.fori_loop(0, c_steps, pad_body, 0)

        def row_body(ci, _):
            cs = pl.ds(ci * _CC, _CC)
            px2_ref[cs, 0, :] = px2_ref[cs, 4, :]
            px2_ref[cs, 1, :] = px2_ref[cs, 3, :]
            px2_ref[cs, h + 2, :] = px2_ref[cs, h, :]
            px2_ref[cs, h + 3, :] = px2_ref[cs, h - 1, :]
            return 0

        lax.fori_loop(0, c_steps, row_body, 0)

    r0 = s * _BR
    zero = jnp.zeros((_BR, w), dtype=jnp.float32)

    def e_body(ci, carry):
        cs = pl.ds(ci * _CC, _CC)
        x1c = x1_ref[0, cs, :, :].astype(jnp.bfloat16)
        win = px2_ref[cs, pl.ds(r0, _BR + 8), :]
        new = list(carry)
        for o, (dr, dc) in enumerate(_OFFS):
            sh = win[:, dr:dr + _BR, dc:dc + w]
            part = jnp.sum(x1c * sh, axis=0)
            new[o] = new[o] + part.astype(jnp.float32)
        return tuple(new)

    es = lax.fori_loop(0, c_steps, e_body, tuple(zero for _ in _OFFS))

    m = es[0]
    for o in range(1, 25):
        m = jnp.maximum(m, es[o])
    exps = [jnp.exp(e - m) for e in es]
    tot = exps[0]
    for o in range(1, 25):
        tot = tot + exps[o]
    inv = 1.0 / tot
    wts = [(e * inv).astype(jnp.bfloat16) for e in exps]

    alpha = alpha_ref[0]
    msk = mask_ref[...]

    def f_body(ci, _):
        cs = pl.ds(ci * _CC, _CC)
        x1c = x1_ref[0, cs, :, :]
        acc = jnp.zeros((x1c.shape[0], _BR, w), dtype=jnp.bfloat16)
        win = px2_ref[cs, pl.ds(r0, _BR + 8), :]
        for o, (dr, dc) in enumerate(_OFFS):
            sh = win[:, dr:dr + _BR, dc:dc + w]
            acc = acc + wts[o][None] * sh
        xf = x1c + msk[None] * (acc.astype(jnp.float32) - x1c)
        out_ref[0, cs, :, :] = x1c + alpha * xf
        return 0

    lax.fori_loop(0, c_steps, f_body, 0)


@jax.jit
def kernel(x1, x2, alpha):
    n, c, h, w = x1.shape
    SFL = _sample_points(h, w, n, _P)
    ph = SFL[0, :, 0]
    pw = SFL[0, :, 1]
    mask = jnp.zeros((h, w), jnp.float32).at[ph, pw].set(1.0)

    grid = (n, h // _BR)
    out = pl.pallas_call(
        _attn_kernel,
        grid=grid,
        in_specs=[
            pl.BlockSpec(memory_space=pltpu.SMEM),
            pl.BlockSpec((_BR, w), lambda b, s: (s, 0)),
            pl.BlockSpec((1, c, _BR, w), lambda b, s: (b, 0, s, 0)),
            pl.BlockSpec((1, c, h, w), lambda b, s: (b, 0, 0, 0)),
        ],
        out_specs=pl.BlockSpec((1, c, _BR, w), lambda b, s: (b, 0, s, 0)),
        out_shape=jax.ShapeDtypeStruct((n, c, h, w), jnp.float32),
        scratch_shapes=[pltpu.VMEM((c, h + 8, w + 4), jnp.bfloat16)],
        compiler_params=pltpu.CompilerParams(
            dimension_semantics=("arbitrary", "arbitrary"),
        ),
    )(alpha, mask, x1, x2.astype(jnp.bfloat16))
    return (out, SFL)
